# baseline (speedup 1.0000x reference)
import jax
import jax.numpy as jnp
import numpy as np
from jax.experimental import pallas as pl
from jax.experimental.pallas import tpu as pltpu

N = 10000
IN_F = 256
OUT_F = 256
JOINT = 3 * OUT_F

_PERM = np.concatenate([np.arange(k, JOINT, 3) for k in range(3)])

_BK = 2048
_NK = 5
_BM = 1000
_NM = N // _BM
_UPFRONT = _BK
_PIECE = 200
_NPIECE = (N - _UPFRONT) // _PIECE
_LAST_PIECE = (N - _UPFRONT) - _NPIECE * _PIECE
_SUP_PAD = _NK * _BK - N


def _fused_body(x_ref, w_ref, bvec_ref, adj_ref, bias_ref, out_ref,
                sup_ref, acc_ref):
    k = pl.program_id(0)
    m = pl.program_id(1)

    def stage_a(base, nrows):
        base = pl.multiple_of(base, 8)
        x = x_ref[pl.ds(base, nrows), :].astype(jnp.bfloat16)
        ab = jnp.dot(x, w_ref[...], preferred_element_type=jnp.float32) + bvec_ref[...]
        s = ab[:, :JOINT] * ab[:, JOINT:]
        iq = s[:, :OUT_F] + s[:, OUT_F:2 * OUT_F] + s[:, 2 * OUT_F:]
        absiq = jnp.abs(iq)
        ssq = jnp.sum(absiq, axis=1, keepdims=True)
        sgn_sqrt = iq * jax.lax.rsqrt(jnp.maximum(absiq, 1e-30))
        rnorm = jax.lax.rsqrt(jnp.maximum(ssq, 1e-24))
        sup_ref[pl.ds(base, nrows), :] = sgn_sqrt * rnorm

    @pl.when((k == 0) & (m == 0))
    def _():
        stage_a(0, _UPFRONT // 2)
        stage_a(_UPFRONT // 2, _UPFRONT // 2)
        sup_ref[pl.ds(N, _SUP_PAD), :] = jnp.zeros((_SUP_PAD, OUT_F), jnp.float32)

    s_idx = k * _NM + m

    @pl.when((k < _NK - 1) & (s_idx < _NPIECE))
    def _():
        stage_a(_UPFRONT + s_idx * _PIECE, _PIECE)

    @pl.when((k < _NK - 1) & (s_idx == _NPIECE))
    def _():
        stage_a(_UPFRONT + _NPIECE * _PIECE, _LAST_PIECE)

    part = jax.lax.dot_general(
        adj_ref[...], sup_ref[pl.ds(pl.multiple_of(k * _BK, 8), _BK), :],
        dimension_numbers=(((1,), (0,)), ((), ())),
        precision=jax.lax.Precision.DEFAULT,
        preferred_element_type=jnp.float32,
    )

    mbase = pl.multiple_of(m * _BM, 8)

    @pl.when(k == 0)
    def _():
        acc_ref[pl.ds(mbase, _BM), :] = part

    @pl.when(k > 0)
    def _():
        acc_ref[pl.ds(mbase, _BM), :] = acc_ref[pl.ds(mbase, _BM), :] + part

    @pl.when(k == _NK - 1)
    def _():
        out_ref[...] = acc_ref[pl.ds(mbase, _BM), :] + bias_ref[...]


def kernel(input, adj, W1, b1, W2, b2, bias):
    wcat = jnp.concatenate(
        [W1[:, _PERM], W2[:, _PERM]], axis=1).astype(jnp.bfloat16)
    bcat = jnp.concatenate([b1[_PERM], b2[_PERM]]).reshape(1, 2 * JOINT)

    return pl.pallas_call(
        _fused_body,
        grid=(_NK, _NM),
        in_specs=[
            pl.BlockSpec((N, IN_F), lambda k, m: (0, 0)),
            pl.BlockSpec((IN_F, 2 * JOINT), lambda k, m: (0, 0)),
            pl.BlockSpec((1, 2 * JOINT), lambda k, m: (0, 0)),
            pl.BlockSpec((_BM, _BK), lambda k, m: (m, k)),
            pl.BlockSpec((1, OUT_F), lambda k, m: (0, 0)),
        ],
        out_specs=pl.BlockSpec(
            (_BM, OUT_F), lambda k, m: (jnp.where(k == _NK - 1, m, 0), 0)),
        out_shape=jax.ShapeDtypeStruct((N, OUT_F), jnp.float32),
        scratch_shapes=[
            pltpu.VMEM((N + _SUP_PAD, OUT_F), jnp.float32),
            pltpu.VMEM((N, OUT_F), jnp.float32),
        ],
        compiler_params=pltpu.CompilerParams(
            dimension_semantics=("arbitrary", "arbitrary"),
        ),
    )(input, wcat, bcat, adj, bias.reshape(1, OUT_F))


# speedup vs baseline: 1.0258x; 1.0258x over previous
import jax
import jax.numpy as jnp
import numpy as np
from jax.experimental import pallas as pl
from jax.experimental.pallas import tpu as pltpu

N = 10000
IN_F = 256
OUT_F = 256
JOINT = 3 * OUT_F

_PERM = np.concatenate([np.arange(k, JOINT, 3) for k in range(3)])

_BM_A = 1000
_BM_B = 400


def _fused_body(x_ref, w_ref, bvec_ref, adj_ref, bias_ref, out_ref, sup_ref):
    i = pl.program_id(0)

    @pl.when(i == 0)
    def _():
        def chunk(c, carry):
            x = x_ref[pl.ds(c * _BM_A, _BM_A), :].astype(jnp.bfloat16)
            ab = jnp.dot(x, w_ref[...], preferred_element_type=jnp.float32) + bvec_ref[...]
            s = ab[:, :JOINT] * ab[:, JOINT:]
            iq = s[:, :OUT_F] + s[:, OUT_F:2 * OUT_F] + s[:, 2 * OUT_F:]
            absiq = jnp.abs(iq)
            ssq = jnp.sum(absiq, axis=1, keepdims=True)
            sgn_sqrt = iq * jax.lax.rsqrt(jnp.maximum(absiq, 1e-30))
            rnorm = jax.lax.rsqrt(jnp.maximum(ssq, 1e-24))
            sup_ref[pl.ds(c * _BM_A, _BM_A), :] = sgn_sqrt * rnorm
            return carry
        jax.lax.fori_loop(0, N // _BM_A, chunk, 0)

    out_ref[...] = (
        jax.lax.dot_general(
            adj_ref[...], sup_ref[...],
            dimension_numbers=(((1,), (0,)), ((), ())),
            precision=jax.lax.Precision.DEFAULT,
            preferred_element_type=jnp.float32,
        )
        + bias_ref[...]
    )


def kernel(input, adj, W1, b1, W2, b2, bias):
    wcat = jnp.concatenate(
        [W1[:, _PERM], W2[:, _PERM]], axis=1).astype(jnp.bfloat16)
    bcat = jnp.concatenate([b1[_PERM], b2[_PERM]]).reshape(1, 2 * JOINT)

    return pl.pallas_call(
        _fused_body,
        grid=(N // _BM_B,),
        in_specs=[
            pl.BlockSpec((N, IN_F), lambda i: (0, 0)),
            pl.BlockSpec((IN_F, 2 * JOINT), lambda i: (0, 0)),
            pl.BlockSpec((1, 2 * JOINT), lambda i: (0, 0)),
            pl.BlockSpec((_BM_B, N), lambda i: (i, 0)),
            pl.BlockSpec((1, OUT_F), lambda i: (0, 0)),
        ],
        out_specs=pl.BlockSpec((_BM_B, OUT_F), lambda i: (i, 0)),
        out_shape=jax.ShapeDtypeStruct((N, OUT_F), jnp.float32),
        scratch_shapes=[pltpu.VMEM((N, OUT_F), jnp.float32)],
        compiler_params=pltpu.CompilerParams(
            dimension_semantics=("arbitrary",),
        ),
    )(input, wcat, bcat, adj, bias.reshape(1, OUT_F))
